# Initial kernel scaffold; baseline (speedup 1.0000x reference)
import jax
import jax.numpy as jnp
from jax.experimental import pallas as pl


def kernel(inputs, keys, vals):
    raise NotImplementedError("write your pallas kernel here")


# speedup vs baseline: 51.8895x; 51.8895x over previous
import functools

import jax
import jax.numpy as jnp
from jax import lax
from jax.experimental import pallas as pl
from jax.experimental.pallas import tpu as pltpu
from jax.experimental.pallas import tpu_sc as plsc

_BATCH = 16384
_FIELDS = 26
_N = _BATCH * _FIELDS
_TBL = 1216
_NKEYS = 1000
_NC, _NS, _L = 2, 16, 16
_NW = _NC * _NS
_CHUNK = _N // _NW

_mesh = plsc.VectorSubcoreMesh(core_axis_name="c", subcore_axis_name="s")


@functools.partial(
    pl.kernel,
    mesh=_mesh,
    compiler_params=pltpu.CompilerParams(needs_layout_passes=False),
    out_type=jax.ShapeDtypeStruct((_N,), jnp.int32),
    scratch_types=[
        pltpu.VMEM((_TBL,), jnp.int32),
        pltpu.VMEM((_NKEYS,), jnp.int32),
        pltpu.VMEM((_NKEYS,), jnp.int32),
        pltpu.VMEM((_CHUNK,), jnp.int32),
        pltpu.VMEM((_CHUNK,), jnp.int32),
    ],
)
def _lookup(in_hbm, keys_hbm, vals_hbm, out_hbm,
            table_v, keys_v, vals_v, in_v, out_v):
    wid = lax.axis_index("s") * _NC + lax.axis_index("c")
    base = wid * _CHUNK

    pltpu.sync_copy(in_hbm.at[pl.ds(base, _CHUNK)], in_v)
    pltpu.sync_copy(keys_hbm, keys_v)
    pltpu.sync_copy(vals_hbm, vals_v)

    zero = jnp.zeros((_L,), jnp.int32)

    def zbody(i, carry):
        table_v[pl.ds(i * _L, _L)] = zero
        return carry

    lax.fori_loop(0, _TBL // _L, zbody, 0, unroll=8)

    def sbody(i, carry):
        k = keys_v[pl.ds(i * _L, _L)]
        v = vals_v[pl.ds(i * _L, _L)]
        plsc.store_scatter(table_v, [k], v)
        return carry

    lax.fori_loop(0, _NKEYS // _L, sbody, 0, unroll=4)
    if _NKEYS % _L:
        k = keys_v[pl.ds(_NKEYS - _L, _L)]
        v = vals_v[pl.ds(_NKEYS - _L, _L)]
        plsc.store_scatter(table_v, [k], v)

    def gbody(i, carry):
        x = in_v[pl.ds(i * _L, _L)]
        out_v[pl.ds(i * _L, _L)] = plsc.load_gather(table_v, [x])
        return carry

    lax.fori_loop(0, _CHUNK // _L, gbody, 0, unroll=8)

    pltpu.sync_copy(out_v, out_hbm.at[pl.ds(base, _CHUNK)])


@jax.jit
def kernel(inputs, keys, vals):
    looked = _lookup(inputs.reshape(_N), keys, vals)
    values = looked.reshape(_BATCH, _FIELDS)
    mask = inputs != 0
    return values, mask
